# baseline (speedup 1.0000x reference)
import jax
import jax.numpy as jnp
from jax import lax
from jax.experimental import pallas as pl
from jax.experimental.pallas import tpu as pltpu
from jax.experimental.pallas import tpu_sc as plsc
import functools

BATCH = 16384
HIST = 50
EMBED_DIM = 32

NC = 2
NS = 16
NW = NC * NS

LANES = 128
NBLOCKS = HIST * (BATCH // LANES)
BLOCKS_PER_W = NBLOCKS // NW
TC_PER_H = BATCH // LANES
ETILES = EMBED_DIM // 8


def _make_sc_kernel():
  mesh = plsc.VectorSubcoreMesh(
      core_axis_name="c", subcore_axis_name="s",
      num_cores=NC, num_subcores=NS)

  @functools.partial(
      pl.kernel,
      out_type=jax.ShapeDtypeStruct((HIST, ETILES, TC_PER_H, 8, LANES),
                                    jnp.float32),
      mesh=mesh,
      scratch_types=[
          pltpu.VMEM((BLOCKS_PER_W, LANES), jnp.int32),
          pltpu.VMEM((2, LANES, EMBED_DIM), jnp.float32),
          pltpu.VMEM((2, ETILES, 8, LANES), jnp.float32),
          pltpu.SemaphoreType.DMA,
          pltpu.SemaphoreType.DMA,
          pltpu.SemaphoreType.DMA,
          pltpu.SemaphoreType.DMA,
      ],
      compiler_params=pltpu.CompilerParams(use_tc_tiling_on_sc=False,
                                           needs_layout_passes=False),
  )
  def sc_embed(idx_hbm, table_hbm, out_hbm, idx_v, rows_v, trv, g0, g1, s0, s1):
    gsem = [g0, g1]
    ssem = [s0, s1]
    wid = lax.axis_index("s") * NC + lax.axis_index("c")
    base = wid * BLOCKS_PER_W

    pltpu.sync_copy(idx_hbm.at[pl.ds(base, BLOCKS_PER_W)], idx_v)

    lane = lax.iota(jnp.int32, 16)

    def fire_gather(g, b):
      pltpu.async_copy(table_hbm.at[idx_v.at[g]], rows_v.at[b], gsem[b])

    def wait_gather(b):
      pltpu.make_async_copy(table_hbm.at[idx_v.at[0]], rows_v.at[b],
                            gsem[b]).wait()

    def transpose(b):
      @plsc.parallel_loop(0, 8, unroll=4)
      def _(blk):
        rid = lane + blk * 16
        for e in range(EMBED_DIM):
          v = plsc.load_gather(
              rows_v.at[b], [rid, jnp.full((16,), e, jnp.int32)])
          trv[b, e // 8, e % 8, pl.ds(blk * 16, 16)] = v

    def fire_stores(g, b):
      h = g // TC_PER_H
      tc = g % TC_PER_H
      for tr in range(ETILES):
        pltpu.async_copy(trv.at[b, tr], out_hbm.at[h, tr, tc], ssem[b])

    def wait_stores(g, b):
      h = g // TC_PER_H
      tc = g % TC_PER_H
      for tr in range(ETILES):
        pltpu.make_async_copy(trv.at[b, tr], out_hbm.at[h, tr, tc],
                              ssem[b]).wait()

    fire_gather(0, 0)
    fire_gather(1, 1)

    for bsel in range(2):
      wait_gather(bsel)
      transpose(bsel)
      fire_gather(2 + bsel, bsel)
      fire_stores(base + bsel, bsel)

    @pl.loop(1, BLOCKS_PER_W // 2 - 1)
    def _(i):
      for bsel in range(2):
        gl = i * 2 + bsel
        g = base + gl
        wait_gather(bsel)
        wait_stores(g - 2, bsel)
        transpose(bsel)
        fire_gather(gl + 2, bsel)
        fire_stores(g, bsel)

    for bsel in range(2):
      g = base + BLOCKS_PER_W - 2 + bsel
      wait_gather(bsel)
      wait_stores(g - 2, bsel)
      transpose(bsel)
      fire_stores(g, bsel)

    wait_stores(base + BLOCKS_PER_W - 2, 0)
    wait_stores(base + BLOCKS_PER_W - 1, 1)

  return sc_embed


def kernel(x, table):
  idx = x.T.reshape(NBLOCKS, LANES).astype(jnp.int32)
  out5 = _make_sc_kernel()(idx, table)
  return out5.transpose(2, 4, 0, 1, 3).reshape(BATCH, HIST, EMBED_DIM)


# speedup vs baseline: 1.0373x; 1.0373x over previous
import jax
import jax.numpy as jnp
from jax import lax
from jax.experimental import pallas as pl
from jax.experimental.pallas import tpu as pltpu
from jax.experimental.pallas import tpu_sc as plsc
import functools

BATCH = 16384
HIST = 50
EMBED_DIM = 32

NC = 2
NS = 16
NW = NC * NS

LANES = 128
NBLOCKS = HIST * (BATCH // LANES)
BLOCKS_PER_W = NBLOCKS // NW
TC_PER_H = BATCH // LANES
ETILES = EMBED_DIM // 8


def _make_sc_kernel():
  mesh = plsc.VectorSubcoreMesh(
      core_axis_name="c", subcore_axis_name="s",
      num_cores=NC, num_subcores=NS)

  @functools.partial(
      pl.kernel,
      out_type=jax.ShapeDtypeStruct((HIST, ETILES, TC_PER_H, 8, LANES),
                                    jnp.float32),
      mesh=mesh,
      scratch_types=[
          pltpu.VMEM((BLOCKS_PER_W, LANES), jnp.int32),
          pltpu.VMEM((2, LANES, EMBED_DIM), jnp.float32),
          pltpu.VMEM((2, ETILES, 8, LANES), jnp.float32),
          pltpu.SemaphoreType.DMA,
          pltpu.SemaphoreType.DMA,
          pltpu.SemaphoreType.DMA,
          pltpu.SemaphoreType.DMA,
      ],
      compiler_params=pltpu.CompilerParams(use_tc_tiling_on_sc=False,
                                           needs_layout_passes=False),
  )
  def sc_embed(idx_hbm, table_hbm, out_hbm, idx_v, rows_v, trv, g0, g1, s0, s1):
    gsem = [g0, g1]
    ssem = [s0, s1]
    wid = lax.axis_index("s") * NC + lax.axis_index("c")
    base = wid * BLOCKS_PER_W

    pltpu.sync_copy(idx_hbm.at[pl.ds(base, BLOCKS_PER_W)], idx_v)

    lane = lax.iota(jnp.int32, 16)

    def fire_gather(g, b):
      pltpu.async_copy(table_hbm.at[idx_v.at[g]], rows_v.at[b], gsem[b])

    def wait_gather(b):
      pltpu.make_async_copy(table_hbm.at[idx_v.at[0]], rows_v.at[b],
                            gsem[b]).wait()

    def transpose(b):
      @plsc.parallel_loop(0, 8, unroll=2)
      def _(blk):
        rid = lane + blk * 16
        for e in range(EMBED_DIM):
          v = plsc.load_gather(
              rows_v.at[b], [rid, jnp.full((16,), e, jnp.int32)])
          trv[b, e // 8, e % 8, pl.ds(blk * 16, 16)] = v

    def fire_stores(g, b):
      h = g // TC_PER_H
      tc = g % TC_PER_H
      for tr in range(ETILES):
        pltpu.async_copy(trv.at[b, tr], out_hbm.at[h, tr, tc], ssem[b])

    def wait_stores(g, b):
      h = g // TC_PER_H
      tc = g % TC_PER_H
      for tr in range(ETILES):
        pltpu.make_async_copy(trv.at[b, tr], out_hbm.at[h, tr, tc],
                              ssem[b]).wait()

    fire_gather(0, 0)
    fire_gather(1, 1)

    for bsel in range(2):
      wait_gather(bsel)
      transpose(bsel)
      fire_gather(2 + bsel, bsel)
      fire_stores(base + bsel, bsel)

    @pl.loop(1, BLOCKS_PER_W // 2 - 1)
    def _(i):
      for bsel in range(2):
        gl = i * 2 + bsel
        g = base + gl
        wait_gather(bsel)
        wait_stores(g - 2, bsel)
        transpose(bsel)
        fire_gather(gl + 2, bsel)
        fire_stores(g, bsel)

    for bsel in range(2):
      g = base + BLOCKS_PER_W - 2 + bsel
      wait_gather(bsel)
      wait_stores(g - 2, bsel)
      transpose(bsel)
      fire_stores(g, bsel)

    wait_stores(base + BLOCKS_PER_W - 2, 0)
    wait_stores(base + BLOCKS_PER_W - 1, 1)

  return sc_embed


def kernel(x, table):
  idx = x.T.reshape(NBLOCKS, LANES).astype(jnp.int32)
  out5 = _make_sc_kernel()(idx, table)
  return out5.transpose(2, 4, 0, 1, 3).reshape(BATCH, HIST, EMBED_DIM)
